# baseline (speedup 1.0000x reference)
import jax
import jax.numpy as jnp
import numpy as np
from jax.experimental import pallas as pl
from jax.experimental.pallas import tpu as pltpu

_VOCAB = 4096
_B = 64
_C = 32
_HW = 16
_S = _HW * _HW
_N_FULL = _B * _S
_PNS = (1, 2, 4, 8, 16)
_PI = (0, 1, 1, 2)
_F32 = jnp.float32
_BF16 = jnp.bfloat16
_HIGH = jax.lax.Precision.HIGHEST
_CHUNK = 512
_CONV_ROWS = 2048
_NT = (((1,), (1,)), ((), ()))


def _pool_mat_t(pn: int) -> np.ndarray:
    k = _HW // pn
    p1 = np.zeros((pn, _HW), np.float32)
    for p in range(pn):
        p1[p, p * k:(p + 1) * k] = 1.0 / k
    p2 = np.kron(p1, p1)
    out = np.zeros((_S, 64), np.float32)
    out[:, :pn * pn] = p2.T
    return out


def _upsample_stack(pn: int):
    a = jax.image.resize(jnp.eye(pn, dtype=_F32), (_HW, pn), method="bicubic")
    ut = jnp.kron(a, a).T
    hi = ut.astype(_BF16)
    lo = (ut - hi.astype(_F32)).astype(_BF16)
    sq = pn * pn
    out = jnp.zeros((192, _S), _BF16)
    out = out.at[0:sq, :].set(hi)
    out = out.at[sq:2 * sq, :].set(lo)
    out = out.at[2 * sq:3 * sq, :].set(hi)
    return out


def _hw_of_cf(x_cf):
    return jnp.swapaxes(x_cf.reshape(_B, _C, _S), 1, 2).reshape(_N_FULL, _C)


def _cf_of_hw(x_hw):
    return jnp.swapaxes(x_hw.reshape(_B, _S, _C), 1, 2).reshape(_B * _C, _S)


def _nc_of_cf(x_cf, sq):
    return jnp.swapaxes(x_cf.reshape(_B, _C, sq), 1, 2).reshape(_B * sq, _C)


def _cf_of_nc(x_nc, sq):
    return jnp.swapaxes(x_nc.reshape(_B, sq, _C), 1, 2).reshape(_B * _C, sq)


def _hilo(x):
    hi = x.astype(_BF16)
    lo = (x - hi.astype(_F32)).astype(_BF16)
    return hi, lo


def _quant_chunk(rest, b96v, hilo, want_h):
    rh, rl = _hilo(rest)
    a96 = jnp.concatenate([rh, rh, rl], axis=1)
    sims = jax.lax.dot_general(a96, b96v, _NT, preferred_element_type=_F32)
    m = jnp.max(sims, axis=1, keepdims=True)
    h = None
    if want_h:
        onehot = (sims == m).astype(_BF16)
        hl = jax.lax.dot(onehot, hilo, preferred_element_type=_F32)
        h = hl[:, :_C] + hl[:, _C:]
        part = jnp.sum(onehot.astype(_F32), axis=0, keepdims=True)
    else:
        part = jnp.sum(jnp.where(sims == m, 1.0, 0.0), axis=0, keepdims=True)
    return h, part


def _conv_block(x, fr, w9, bias):
    n = x.shape[0]
    zeros = jnp.zeros_like(x)
    ii = jax.lax.broadcasted_iota(jnp.int32, (n, 1), 0)
    w = ii & (_HW - 1)
    h = (ii >> 4) & (_HW - 1)
    taps = []
    for dh in (-1, 0, 1):
        for dw in (-1, 0, 1):
            s = dh * _HW + dw
            if s == 0:
                xs = x
            elif s > 0:
                xs = jnp.concatenate([x[s:, :], zeros[:s, :]], axis=0)
            else:
                xs = jnp.concatenate([zeros[:(-s), :], x[:s, :]], axis=0)
            okh = jnp.logical_and(h + dh >= 0, h + dh < _HW)
            okw = jnp.logical_and(w + dw >= 0, w + dw < _HW)
            taps.append(jnp.where(jnp.logical_and(okh, okw), xs, 0.0)
                        .astype(_BF16))
    y = jnp.broadcast_to(bias, (n, _C)).astype(_F32)
    y = y + jax.lax.dot(jnp.concatenate(taps[:6], axis=1), w9[:6 * _C, :],
                        preferred_element_type=_F32)
    y = y + jax.lax.dot(jnp.concatenate(taps[6:], axis=1), w9[6 * _C:, :],
                        preferred_element_type=_F32)
    return fr - 0.5 * x - 0.5 * y


def _body(f_cf_ref, f_hw_ref, emb_ref, w9s_ref, bias_ref, pts_ref, uts_ref,
          fhat_ref, loss_ref, ppl_ref,
          fr_hw, sc_h, sc_hits):
    emb = emb_ref[...]
    norm = jnp.sqrt(jnp.sum(emb * emb, axis=1, keepdims=True))
    en = emb / jnp.maximum(norm, 1e-12)
    ehi, elo = _hilo(en)
    b96v = jnp.concatenate([ehi, elo, ehi], axis=1)
    ghi, glo = _hilo(emb)
    hilo = jnp.concatenate([ghi, glo], axis=1)

    fr_hw[...] = f_hw_ref[...]
    sc_hits[...] = jnp.zeros_like(sc_hits)

    for si, pn in enumerate(_PNS):
        sq = pn * pn
        n = _B * sq
        last = si == len(_PNS) - 1

        if last:
            def _qloop4(i, _):
                rest = fr_hw[pl.ds(i * _CHUNK, _CHUNK), :]
                _, part = _quant_chunk(rest, b96v, hilo, want_h=False)
                sc_hits[0:1, :] += part
                return 0
            jax.lax.fori_loop(0, _N_FULL // _CHUNK, _qloop4, 0)
            break

        fr_cf = f_cf_ref[...] if si == 0 else _cf_of_hw(fr_hw[...])
        psq = 4 if si == 0 else sq
        pt = pts_ref[si * 2 * _S:(si + 1) * 2 * _S, 0:psq]
        fh, fl = _hilo(fr_cf)
        a512 = jnp.concatenate([fh, fl], axis=1)
        pooled_nc = _nc_of_cf(
            jax.lax.dot(a512, pt, preferred_element_type=_F32), psq)
        if pn == 1:
            r64 = jax.lax.broadcasted_iota(jnp.int32, (_B, 4 * _B), 0)
            c64 = jax.lax.broadcasted_iota(jnp.int32, (_B, 4 * _B), 1)
            m64 = jnp.where(c64 // 4 == r64, 0.25, 0.0)
            rest_nc = jax.lax.dot(m64, pooled_nc, precision=_HIGH)
        else:
            rest_nc = pooled_nc

        if n <= _CHUNK:
            h_nc, part = _quant_chunk(rest_nc, b96v, hilo, want_h=True)
            sc_hits[0:1, :] += part
        else:
            sc_h[pl.ds(0, n), :] = rest_nc

            def _qloop3(i, _):
                rest = sc_h[pl.ds(i * _CHUNK, _CHUNK), :]
                h, part = _quant_chunk(rest, b96v, hilo, want_h=True)
                sc_h[pl.ds(i * _CHUNK, _CHUNK), :] = h
                sc_hits[0:1, :] += part
                return 0
            jax.lax.fori_loop(0, n // _CHUNK, _qloop3, 0)
            h_nc = sc_h[pl.ds(0, n), :]

        if pn == 1:
            hup_hw = jnp.broadcast_to(
                h_nc.reshape(_B, 1, _C), (_B, _S, _C)).reshape(_N_FULL, _C)
        else:
            ut = uts_ref[si * 192:si * 192 + 3 * sq, :]
            h_cf = _cf_of_nc(h_nc, sq)
            hh, hlo = _hilo(h_cf)
            a3 = jnp.concatenate([hh, hh, hlo], axis=1)
            hup_cf = jax.lax.dot(a3, ut, preferred_element_type=_F32)
            hup_hw = _hw_of_cf(hup_cf)
        sc_h[...] = hup_hw

        w9 = w9s_ref[si * 9 * _C:(si + 1) * 9 * _C, :].astype(_BF16)
        bias = bias_ref[si * 8:si * 8 + 1, :]

        def _cloop(j, _):
            rows = pl.ds(j * _CONV_ROWS, _CONV_ROWS)
            fr_hw[rows, :] = _conv_block(sc_h[rows, :], fr_hw[rows, :],
                                         w9, bias)
            return 0
        jax.lax.fori_loop(0, _N_FULL // _CONV_ROWS, _cloop, 0)

    hits = sc_hits[0:1, :]
    total = jnp.sum(hits)
    avg = hits / jnp.maximum(total, 1.0)
    ent = jnp.sum(avg * jnp.log(avg + 1e-10))
    f_cf = f_cf_ref[...]
    loss = 6.25 * (jnp.sum(f_cf * f_cf) / (_B * _C * _S))
    fhat_ref[...] = jnp.zeros_like(fhat_ref)
    loss_ref[...] = jnp.full((8, 128), loss, _F32)
    ppl_ref[...] = jnp.full((8, 128), jnp.exp(-ent), _F32)


def kernel(f_BChw, emb_weight, phi_w, phi_b):
    f = f_BChw.astype(_F32)
    f_cf = f.reshape(_B * _C, _S)
    f_hw = f.transpose(0, 2, 3, 1).reshape(_N_FULL, _C)

    pts = jnp.asarray(np.concatenate(
        [np.concatenate([_pool_mat_t(pn), _pool_mat_t(pn)], axis=0)
         for pn in (2, 2, 4, 8)], axis=0)).astype(_BF16)
    uts = jnp.concatenate([_upsample_stack(pn) for pn in (1, 2, 4, 8)],
                          axis=0)
    w9s = jnp.concatenate(
        [jnp.concatenate([phi_w[_PI[si], :, :, a, b].T
                          for a in range(3) for b in range(3)], axis=0)
         for si in range(4)], axis=0)
    bias = jnp.concatenate(
        [jnp.broadcast_to(phi_b[_PI[si]][None, :], (8, _C))
         for si in range(4)], axis=0)

    fhat_cf, loss_t, ppl_t = pl.pallas_call(
        _body,
        out_shape=[jax.ShapeDtypeStruct((_B * _C, _S), _F32),
                   jax.ShapeDtypeStruct((8, 128), _F32),
                   jax.ShapeDtypeStruct((8, 128), _F32)],
        scratch_shapes=[pltpu.VMEM((_N_FULL, _C), _F32),
                        pltpu.VMEM((_N_FULL, _C), _F32),
                        pltpu.VMEM((8, _VOCAB), _F32)],
    )(f_cf, f_hw, emb_weight, w9s, bias, pts, uts)

    return (fhat_cf.reshape(_B, _C, _HW, _HW), loss_t[0, 0], ppl_t[0, 0])


# speedup vs baseline: 1.0691x; 1.0691x over previous
import jax
import jax.numpy as jnp
import numpy as np
from jax.experimental import pallas as pl
from jax.experimental.pallas import tpu as pltpu

_VOCAB = 4096
_B = 64
_C = 32
_HW = 16
_S = _HW * _HW
_N_FULL = _B * _S
_PNS = (1, 2, 4, 8, 16)
_PI = (0, 1, 1, 2)
_F32 = jnp.float32
_BF16 = jnp.bfloat16
_HIGH = jax.lax.Precision.HIGHEST
_CHUNK = 256
_CONV_ROWS = 2048
_NT = (((1,), (1,)), ((), ()))


def _pool_mat_t(pn: int) -> np.ndarray:
    k = _HW // pn
    p1 = np.zeros((pn, _HW), np.float32)
    for p in range(pn):
        p1[p, p * k:(p + 1) * k] = 1.0 / k
    p2 = np.kron(p1, p1)
    out = np.zeros((_S, 64), np.float32)
    out[:, :pn * pn] = p2.T
    return out


def _upsample_stack(pn: int):
    a = jax.image.resize(jnp.eye(pn, dtype=_F32), (_HW, pn), method="bicubic")
    ut = jnp.kron(a, a).T
    hi = ut.astype(_BF16)
    lo = (ut - hi.astype(_F32)).astype(_BF16)
    sq = pn * pn
    out = jnp.zeros((192, _S), _BF16)
    out = out.at[0:sq, :].set(hi)
    out = out.at[sq:2 * sq, :].set(lo)
    out = out.at[2 * sq:3 * sq, :].set(hi)
    return out


def _hw_of_cf(x_cf):
    return jnp.swapaxes(x_cf.reshape(_B, _C, _S), 1, 2).reshape(_N_FULL, _C)


def _cf_of_hw(x_hw):
    return jnp.swapaxes(x_hw.reshape(_B, _S, _C), 1, 2).reshape(_B * _C, _S)


def _nc_of_cf(x_cf, sq):
    return jnp.swapaxes(x_cf.reshape(_B, _C, sq), 1, 2).reshape(_B * sq, _C)


def _cf_of_nc(x_nc, sq):
    return jnp.swapaxes(x_nc.reshape(_B, sq, _C), 1, 2).reshape(_B * _C, sq)


def _hilo(x):
    hi = x.astype(_BF16)
    lo = (x - hi.astype(_F32)).astype(_BF16)
    return hi, lo


def _quant_chunk(rest, b96v, hilo, want_h):
    rh, rl = _hilo(rest)
    a96 = jnp.concatenate([rh, rh, rl], axis=1)
    sims = jax.lax.dot_general(a96, b96v, _NT, preferred_element_type=_F32)
    m = jnp.max(sims, axis=1, keepdims=True)
    onehot = (sims == m).astype(_BF16)
    h = None
    if want_h:
        hl = jax.lax.dot(onehot, hilo, preferred_element_type=_F32)
        h = hl[:, :_C] + hl[:, _C:]
    chunk = rest.shape[0]
    rows = jax.lax.broadcasted_iota(jnp.int32, (8, chunk), 0)
    cols = jax.lax.broadcasted_iota(jnp.int32, (8, chunk), 1)
    w8 = (cols // (chunk // 8) == rows).astype(_BF16)
    part = jax.lax.dot(w8, onehot, preferred_element_type=_F32)
    return h, part


def _conv_block(x, fr, w9, bias):
    n = x.shape[0]
    zeros = jnp.zeros_like(x)
    ii = jax.lax.broadcasted_iota(jnp.int32, (n, 1), 0)
    w = ii & (_HW - 1)
    h = (ii >> 4) & (_HW - 1)
    taps = []
    for dh in (-1, 0, 1):
        for dw in (-1, 0, 1):
            s = dh * _HW + dw
            if s == 0:
                xs = x
            elif s > 0:
                xs = jnp.concatenate([x[s:, :], zeros[:s, :]], axis=0)
            else:
                xs = jnp.concatenate([zeros[:(-s), :], x[:s, :]], axis=0)
            okh = jnp.logical_and(h + dh >= 0, h + dh < _HW)
            okw = jnp.logical_and(w + dw >= 0, w + dw < _HW)
            taps.append(jnp.where(jnp.logical_and(okh, okw), xs, 0.0)
                        .astype(_BF16))
    y = jnp.broadcast_to(bias, (n, _C)).astype(_F32)
    y = y + jax.lax.dot(jnp.concatenate(taps[:6], axis=1), w9[:6 * _C, :],
                        preferred_element_type=_F32)
    y = y + jax.lax.dot(jnp.concatenate(taps[6:], axis=1), w9[6 * _C:, :],
                        preferred_element_type=_F32)
    return fr - 0.5 * x - 0.5 * y


def _body(f_cf_ref, f_hw_ref, emb_ref, w9s_ref, bias_ref, pts_ref, uts_ref,
          fhat_ref, loss_ref, ppl_ref,
          fr_hw, sc_h, sc_hits):
    emb = emb_ref[...]
    norm = jnp.sqrt(jnp.sum(emb * emb, axis=1, keepdims=True))
    en = emb / jnp.maximum(norm, 1e-12)
    ehi, elo = _hilo(en)
    b96v = jnp.concatenate([ehi, elo, ehi], axis=1)
    ghi, glo = _hilo(emb)
    hilo = jnp.concatenate([ghi, glo], axis=1)

    fr_hw[...] = f_hw_ref[...]
    sc_hits[...] = jnp.zeros_like(sc_hits)

    for si, pn in enumerate(_PNS):
        sq = pn * pn
        n = _B * sq
        last = si == len(_PNS) - 1

        if last:
            def _qloop4(i, _):
                rest_a = fr_hw[pl.ds(2 * i * _CHUNK, _CHUNK), :]
                rest_b = fr_hw[pl.ds((2 * i + 1) * _CHUNK, _CHUNK), :]
                _, pa = _quant_chunk(rest_a, b96v, hilo, want_h=False)
                _, pb = _quant_chunk(rest_b, b96v, hilo, want_h=False)
                sc_hits[...] += pa + pb
                return 0
            jax.lax.fori_loop(0, _N_FULL // _CHUNK // 2, _qloop4, 0)
            break

        fr_cf = f_cf_ref[...] if si == 0 else _cf_of_hw(fr_hw[...])
        psq = 4 if si == 0 else sq
        pt = pts_ref[si * 2 * _S:(si + 1) * 2 * _S, 0:psq]
        fh, fl = _hilo(fr_cf)
        a512 = jnp.concatenate([fh, fl], axis=1)
        pooled_nc = _nc_of_cf(
            jax.lax.dot(a512, pt, preferred_element_type=_F32), psq)
        if pn == 1:
            r64 = jax.lax.broadcasted_iota(jnp.int32, (_B, 4 * _B), 0)
            c64 = jax.lax.broadcasted_iota(jnp.int32, (_B, 4 * _B), 1)
            m64 = jnp.where(c64 // 4 == r64, 0.25, 0.0)
            rest_nc = jax.lax.dot(m64, pooled_nc, precision=_HIGH)
        else:
            rest_nc = pooled_nc

        if n <= _CHUNK:
            h_nc, part = _quant_chunk(rest_nc, b96v, hilo, want_h=True)
            sc_hits[...] += part
        else:
            sc_h[pl.ds(0, n), :] = rest_nc

            def _qloop3(i, _):
                ra = sc_h[pl.ds(2 * i * _CHUNK, _CHUNK), :]
                rb = sc_h[pl.ds((2 * i + 1) * _CHUNK, _CHUNK), :]
                ha, pa = _quant_chunk(ra, b96v, hilo, want_h=True)
                hb, pb = _quant_chunk(rb, b96v, hilo, want_h=True)
                sc_h[pl.ds(2 * i * _CHUNK, _CHUNK), :] = ha
                sc_h[pl.ds((2 * i + 1) * _CHUNK, _CHUNK), :] = hb
                sc_hits[...] += pa + pb
                return 0
            jax.lax.fori_loop(0, n // _CHUNK // 2, _qloop3, 0)
            h_nc = sc_h[pl.ds(0, n), :]

        if pn == 1:
            hup_hw = jnp.broadcast_to(
                h_nc.reshape(_B, 1, _C), (_B, _S, _C)).reshape(_N_FULL, _C)
        else:
            ut = uts_ref[si * 192:si * 192 + 3 * sq, :]
            h_cf = _cf_of_nc(h_nc, sq)
            hh, hlo = _hilo(h_cf)
            a3 = jnp.concatenate([hh, hh, hlo], axis=1)
            hup_cf = jax.lax.dot(a3, ut, preferred_element_type=_F32)
            hup_hw = _hw_of_cf(hup_cf)
        sc_h[...] = hup_hw

        w9 = w9s_ref[si * 9 * _C:(si + 1) * 9 * _C, :].astype(_BF16)
        bias = bias_ref[si * 8:si * 8 + 1, :]

        def _cloop(j, _):
            rows = pl.ds(j * _CONV_ROWS, _CONV_ROWS)
            fr_hw[rows, :] = _conv_block(sc_h[rows, :], fr_hw[rows, :],
                                         w9, bias)
            return 0
        jax.lax.fori_loop(0, _N_FULL // _CONV_ROWS, _cloop, 0)

    hits = jnp.sum(sc_hits[...], axis=0, keepdims=True)
    total = jnp.sum(hits)
    avg = hits / jnp.maximum(total, 1.0)
    ent = jnp.sum(avg * jnp.log(avg + 1e-10))
    f_cf = f_cf_ref[...]
    loss = 6.25 * (jnp.sum(f_cf * f_cf) / (_B * _C * _S))
    fhat_ref[...] = jnp.zeros_like(fhat_ref)
    loss_ref[...] = jnp.full((8, 128), loss, _F32)
    ppl_ref[...] = jnp.full((8, 128), jnp.exp(-ent), _F32)


def kernel(f_BChw, emb_weight, phi_w, phi_b):
    f = f_BChw.astype(_F32)
    f_cf = f.reshape(_B * _C, _S)
    f_hw = f.transpose(0, 2, 3, 1).reshape(_N_FULL, _C)

    pts = jnp.asarray(np.concatenate(
        [np.concatenate([_pool_mat_t(pn), _pool_mat_t(pn)], axis=0)
         for pn in (2, 2, 4, 8)], axis=0)).astype(_BF16)
    uts = jnp.concatenate([_upsample_stack(pn) for pn in (1, 2, 4, 8)],
                          axis=0)
    w9s = jnp.concatenate(
        [jnp.concatenate([phi_w[_PI[si], :, :, a, b].T
                          for a in range(3) for b in range(3)], axis=0)
         for si in range(4)], axis=0)
    bias = jnp.concatenate(
        [jnp.broadcast_to(phi_b[_PI[si]][None, :], (8, _C))
         for si in range(4)], axis=0)

    fhat_cf, loss_t, ppl_t = pl.pallas_call(
        _body,
        out_shape=[jax.ShapeDtypeStruct((_B * _C, _S), _F32),
                   jax.ShapeDtypeStruct((8, 128), _F32),
                   jax.ShapeDtypeStruct((8, 128), _F32)],
        scratch_shapes=[pltpu.VMEM((_N_FULL, _C), _F32),
                        pltpu.VMEM((_N_FULL, _C), _F32),
                        pltpu.VMEM((8, _VOCAB), _F32)],
    )(f_cf, f_hw, emb_weight, w9s, bias, pts, uts)

    return (fhat_cf.reshape(_B, _C, _HW, _HW), loss_t[0, 0], ppl_t[0, 0])


# speedup vs baseline: 1.0941x; 1.0234x over previous
import jax
import jax.numpy as jnp
import numpy as np
from jax.experimental import pallas as pl
from jax.experimental.pallas import tpu as pltpu

_VOCAB = 4096
_B = 64
_C = 32
_HW = 16
_S = _HW * _HW
_N_FULL = _B * _S
_PNS = (1, 2, 4, 8, 16)
_PI = (0, 1, 1, 2)
_F32 = jnp.float32
_BF16 = jnp.bfloat16
_HIGH = jax.lax.Precision.HIGHEST
_CHUNK = 256
_CONV_ROWS = 2048
_NT = (((1,), (1,)), ((), ()))


def _pool_mat_t(pn: int) -> np.ndarray:
    k = _HW // pn
    p1 = np.zeros((pn, _HW), np.float32)
    for p in range(pn):
        p1[p, p * k:(p + 1) * k] = 1.0 / k
    p2 = np.kron(p1, p1)
    out = np.zeros((_S, 64), np.float32)
    out[:, :pn * pn] = p2.T
    return out


def _upsample_stack(pn: int):
    a = jax.image.resize(jnp.eye(pn, dtype=_F32), (_HW, pn), method="bicubic")
    ut = jnp.kron(a, a).T
    hi = ut.astype(_BF16)
    lo = (ut - hi.astype(_F32)).astype(_BF16)
    sq = pn * pn
    out = jnp.zeros((192, _S), _BF16)
    out = out.at[0:sq, :].set(hi)
    out = out.at[sq:2 * sq, :].set(lo)
    out = out.at[2 * sq:3 * sq, :].set(hi)
    return out


def _hw_of_cf(x_cf):
    return jnp.swapaxes(x_cf.reshape(_B, _C, _S), 1, 2).reshape(_N_FULL, _C)


def _cf_of_hw(x_hw):
    return jnp.swapaxes(x_hw.reshape(_B, _S, _C), 1, 2).reshape(_B * _C, _S)


def _nc_of_cf(x_cf, sq):
    return jnp.swapaxes(x_cf.reshape(_B, _C, sq), 1, 2).reshape(_B * sq, _C)


def _cf_of_nc(x_nc, sq):
    return jnp.swapaxes(x_nc.reshape(_B, sq, _C), 1, 2).reshape(_B * _C, sq)


def _hilo(x):
    hi = x.astype(_BF16)
    lo = (x - hi.astype(_F32)).astype(_BF16)
    return hi, lo


def _quant_chunk(rest, b96v, hilo, want_h):
    rh, rl = _hilo(rest)
    a96 = jnp.concatenate([rh, rh, rl], axis=1)
    sims = jax.lax.dot_general(a96, b96v, _NT, preferred_element_type=_F32)
    m = jnp.max(sims, axis=1, keepdims=True)
    h = None
    if want_h:
        onehot = (sims == m).astype(_BF16)
        hl = jax.lax.dot(onehot, hilo, preferred_element_type=_F32)
        h = hl[:, :_C] + hl[:, _C:]
        part = jnp.sum(onehot.astype(_F32), axis=0, keepdims=True)
    else:
        part = jnp.sum(jnp.where(sims == m, 1.0, 0.0), axis=0, keepdims=True)
    return h, part


def _conv_block(x, fr, w9, bias):
    n = x.shape[0]
    zeros = jnp.zeros_like(x)
    ii = jax.lax.broadcasted_iota(jnp.int32, (n, 1), 0)
    w = ii & (_HW - 1)
    h = (ii >> 4) & (_HW - 1)
    taps = []
    for dh in (-1, 0, 1):
        for dw in (-1, 0, 1):
            s = dh * _HW + dw
            if s == 0:
                xs = x
            elif s > 0:
                xs = jnp.concatenate([x[s:, :], zeros[:s, :]], axis=0)
            else:
                xs = jnp.concatenate([zeros[:(-s), :], x[:s, :]], axis=0)
            okh = jnp.logical_and(h + dh >= 0, h + dh < _HW)
            okw = jnp.logical_and(w + dw >= 0, w + dw < _HW)
            taps.append(jnp.where(jnp.logical_and(okh, okw), xs, 0.0)
                        .astype(_BF16))
    y = jnp.broadcast_to(bias, (n, _C)).astype(_F32)
    y = y + jax.lax.dot(jnp.concatenate(taps[:6], axis=1), w9[:6 * _C, :],
                        preferred_element_type=_F32)
    y = y + jax.lax.dot(jnp.concatenate(taps[6:], axis=1), w9[6 * _C:, :],
                        preferred_element_type=_F32)
    return fr - 0.5 * x - 0.5 * y


def _body(f_cf_ref, f_hw_ref, emb_ref, w9s_ref, bias_ref, pts_ref, uts_ref,
          fhat_ref, loss_ref, ppl_ref,
          fr_hw, sc_h, sc_hits):
    emb = emb_ref[...]
    norm = jnp.sqrt(jnp.sum(emb * emb, axis=1, keepdims=True))
    en = emb / jnp.maximum(norm, 1e-12)
    ehi, elo = _hilo(en)
    b96v = jnp.concatenate([ehi, elo, ehi], axis=1)
    ghi, glo = _hilo(emb)
    hilo = jnp.concatenate([ghi, glo], axis=1)

    fr_hw[...] = f_hw_ref[...]
    sc_hits[...] = jnp.zeros_like(sc_hits)

    for si, pn in enumerate(_PNS):
        sq = pn * pn
        n = _B * sq
        last = si == len(_PNS) - 1

        if last:
            def _qloop4(i, _):
                rest_a = fr_hw[pl.ds(2 * i * _CHUNK, _CHUNK), :]
                rest_b = fr_hw[pl.ds((2 * i + 1) * _CHUNK, _CHUNK), :]
                _, pa = _quant_chunk(rest_a, b96v, hilo, want_h=False)
                _, pb = _quant_chunk(rest_b, b96v, hilo, want_h=False)
                sc_hits[0:1, :] += pa + pb
                return 0
            jax.lax.fori_loop(0, _N_FULL // _CHUNK // 2, _qloop4, 0)
            break

        fr_cf = f_cf_ref[...] if si == 0 else _cf_of_hw(fr_hw[...])
        psq = 4 if si == 0 else sq
        pt = pts_ref[si * 2 * _S:(si + 1) * 2 * _S, 0:psq]
        fh, fl = _hilo(fr_cf)
        a512 = jnp.concatenate([fh, fl], axis=1)
        pooled_nc = _nc_of_cf(
            jax.lax.dot(a512, pt, preferred_element_type=_F32), psq)
        if pn == 1:
            r64 = jax.lax.broadcasted_iota(jnp.int32, (_B, 4 * _B), 0)
            c64 = jax.lax.broadcasted_iota(jnp.int32, (_B, 4 * _B), 1)
            m64 = jnp.where(c64 // 4 == r64, 0.25, 0.0)
            rest_nc = jax.lax.dot(m64, pooled_nc, precision=_HIGH)
        else:
            rest_nc = pooled_nc

        if n <= _CHUNK:
            h_nc, part = _quant_chunk(rest_nc, b96v, hilo, want_h=True)
            sc_hits[0:1, :] += part
        else:
            sc_h[pl.ds(0, n), :] = rest_nc

            def _qloop3(i, _):
                ra = sc_h[pl.ds(2 * i * _CHUNK, _CHUNK), :]
                rb = sc_h[pl.ds((2 * i + 1) * _CHUNK, _CHUNK), :]
                ha, pa = _quant_chunk(ra, b96v, hilo, want_h=True)
                hb, pb = _quant_chunk(rb, b96v, hilo, want_h=True)
                sc_h[pl.ds(2 * i * _CHUNK, _CHUNK), :] = ha
                sc_h[pl.ds((2 * i + 1) * _CHUNK, _CHUNK), :] = hb
                sc_hits[0:1, :] += pa + pb
                return 0
            jax.lax.fori_loop(0, n // _CHUNK // 2, _qloop3, 0)
            h_nc = sc_h[pl.ds(0, n), :]

        if pn == 1:
            hup_hw = jnp.broadcast_to(
                h_nc.reshape(_B, 1, _C), (_B, _S, _C)).reshape(_N_FULL, _C)
        else:
            ut = uts_ref[si * 192:si * 192 + 3 * sq, :]
            h_cf = _cf_of_nc(h_nc, sq)
            hh, hlo = _hilo(h_cf)
            a3 = jnp.concatenate([hh, hh, hlo], axis=1)
            hup_cf = jax.lax.dot(a3, ut, preferred_element_type=_F32)
            hup_hw = _hw_of_cf(hup_cf)
        sc_h[...] = hup_hw

        w9 = w9s_ref[si * 9 * _C:(si + 1) * 9 * _C, :].astype(_BF16)
        bias = bias_ref[si * 8:si * 8 + 1, :]

        def _cloop(j, _):
            rows = pl.ds(j * _CONV_ROWS, _CONV_ROWS)
            fr_hw[rows, :] = _conv_block(sc_h[rows, :], fr_hw[rows, :],
                                         w9, bias)
            return 0
        jax.lax.fori_loop(0, _N_FULL // _CONV_ROWS, _cloop, 0)

    hits = sc_hits[0:1, :]
    total = jnp.sum(hits)
    avg = hits / jnp.maximum(total, 1.0)
    ent = jnp.sum(avg * jnp.log(avg + 1e-10))
    f_cf = f_cf_ref[...]
    loss = 6.25 * (jnp.sum(f_cf * f_cf) / (_B * _C * _S))
    fhat_ref[...] = jnp.zeros_like(fhat_ref)
    loss_ref[...] = jnp.full((8, 128), loss, _F32)
    ppl_ref[...] = jnp.full((8, 128), jnp.exp(-ent), _F32)


def kernel(f_BChw, emb_weight, phi_w, phi_b):
    f = f_BChw.astype(_F32)
    f_cf = f.reshape(_B * _C, _S)
    f_hw = f.transpose(0, 2, 3, 1).reshape(_N_FULL, _C)

    pts = jnp.asarray(np.concatenate(
        [np.concatenate([_pool_mat_t(pn), _pool_mat_t(pn)], axis=0)
         for pn in (2, 2, 4, 8)], axis=0)).astype(_BF16)
    uts = jnp.concatenate([_upsample_stack(pn) for pn in (1, 2, 4, 8)],
                          axis=0)
    w9s = jnp.concatenate(
        [jnp.concatenate([phi_w[_PI[si], :, :, a, b].T
                          for a in range(3) for b in range(3)], axis=0)
         for si in range(4)], axis=0)
    bias = jnp.concatenate(
        [jnp.broadcast_to(phi_b[_PI[si]][None, :], (8, _C))
         for si in range(4)], axis=0)

    fhat_cf, loss_t, ppl_t = pl.pallas_call(
        _body,
        out_shape=[jax.ShapeDtypeStruct((_B * _C, _S), _F32),
                   jax.ShapeDtypeStruct((8, 128), _F32),
                   jax.ShapeDtypeStruct((8, 128), _F32)],
        scratch_shapes=[pltpu.VMEM((_N_FULL, _C), _F32),
                        pltpu.VMEM((_N_FULL, _C), _F32),
                        pltpu.VMEM((8, _VOCAB), _F32)],
    )(f_cf, f_hw, emb_weight, w9s, bias, pts, uts)

    return (fhat_cf.reshape(_B, _C, _HW, _HW), loss_t[0, 0], ppl_t[0, 0])


# speedup vs baseline: 1.1136x; 1.0179x over previous
import jax
import jax.numpy as jnp
import numpy as np
from jax.experimental import pallas as pl
from jax.experimental.pallas import tpu as pltpu

_VOCAB = 4096
_B = 64
_C = 32
_HW = 16
_S = _HW * _HW
_N_FULL = _B * _S
_PNS = (1, 2, 4, 8, 16)
_PI = (0, 1, 1, 2)
_F32 = jnp.float32
_BF16 = jnp.bfloat16
_HIGH = jax.lax.Precision.HIGHEST
_CHUNK = 256
_CONV_ROWS = 2048
_NT = (((1,), (1,)), ((), ()))


def _pool_mat_t(pn: int) -> np.ndarray:
    k = _HW // pn
    p1 = np.zeros((pn, _HW), np.float32)
    for p in range(pn):
        p1[p, p * k:(p + 1) * k] = 1.0 / k
    p2 = np.kron(p1, p1)
    out = np.zeros((_S, 64), np.float32)
    out[:, :pn * pn] = p2.T
    return out


def _upsample_stack(pn: int):
    a = jax.image.resize(jnp.eye(pn, dtype=_F32), (_HW, pn), method="bicubic")
    ut = jnp.kron(a, a).T
    hi = ut.astype(_BF16)
    lo = (ut - hi.astype(_F32)).astype(_BF16)
    sq = pn * pn
    out = jnp.zeros((192, _S), _BF16)
    out = out.at[0:sq, :].set(hi)
    out = out.at[sq:2 * sq, :].set(lo)
    out = out.at[2 * sq:3 * sq, :].set(hi)
    return out


def _hw_of_cf(x_cf):
    return jnp.swapaxes(x_cf.reshape(_B, _C, _S), 1, 2).reshape(_N_FULL, _C)


def _cf_of_hw(x_hw):
    return jnp.swapaxes(x_hw.reshape(_B, _S, _C), 1, 2).reshape(_B * _C, _S)


def _nc_of_cf(x_cf, sq):
    return jnp.swapaxes(x_cf.reshape(_B, _C, sq), 1, 2).reshape(_B * sq, _C)


def _cf_of_nc(x_nc, sq):
    return jnp.swapaxes(x_nc.reshape(_B, sq, _C), 1, 2).reshape(_B * _C, sq)


def _hilo(x):
    hi = x.astype(_BF16)
    lo = (x - hi.astype(_F32)).astype(_BF16)
    return hi, lo


def _quant_chunk(rest, b96v, hilo, want_h):
    rh, rl = _hilo(rest)
    a96 = jnp.concatenate([rh, rh, rl], axis=1)
    sims = jax.lax.dot_general(a96, b96v, _NT, preferred_element_type=_F32)
    m = jnp.max(sims, axis=1, keepdims=True)
    h = None
    if want_h:
        onehot = (sims == m).astype(_BF16)
        hl = jax.lax.dot(onehot, hilo, preferred_element_type=_F32)
        h = hl[:, :_C] + hl[:, _C:]
        part = jnp.sum(onehot.astype(_F32), axis=0, keepdims=True)
    else:
        part = jnp.sum(jnp.where(sims == m, 1.0, 0.0), axis=0, keepdims=True)
    return h, part


def _conv_block(x, fr, w9, bias):
    n = x.shape[0]
    zeros = jnp.zeros_like(x)
    ii = jax.lax.broadcasted_iota(jnp.int32, (n, 1), 0)
    w = ii & (_HW - 1)
    h = (ii >> 4) & (_HW - 1)
    taps = []
    for dh in (-1, 0, 1):
        for dw in (-1, 0, 1):
            s = dh * _HW + dw
            if s == 0:
                xs = x
            elif s > 0:
                xs = jnp.concatenate([x[s:, :], zeros[:s, :]], axis=0)
            else:
                xs = jnp.concatenate([zeros[:(-s), :], x[:s, :]], axis=0)
            okh = jnp.logical_and(h + dh >= 0, h + dh < _HW)
            okw = jnp.logical_and(w + dw >= 0, w + dw < _HW)
            taps.append(jnp.where(jnp.logical_and(okh, okw), xs, 0.0)
                        .astype(_BF16))
    y = jnp.broadcast_to(bias, (n, _C)).astype(_F32)
    y = y + jax.lax.dot(jnp.concatenate(taps[:6], axis=1), w9[:6 * _C, :],
                        preferred_element_type=_F32)
    y = y + jax.lax.dot(jnp.concatenate(taps[6:], axis=1), w9[6 * _C:, :],
                        preferred_element_type=_F32)
    return fr - 0.5 * x - 0.5 * y


def _body(f_cf_ref, f_hw_ref, emb_ref, w9s_ref, bias_ref, pts_ref, uts_ref,
          fhat_ref, loss_ref, ppl_ref,
          fr_hw, sc_h, sc_hits):
    emb = emb_ref[...]
    norm = jnp.sqrt(jnp.sum(emb * emb, axis=1, keepdims=True))
    en = emb / jnp.maximum(norm, 1e-12)
    ehi, elo = _hilo(en)
    b96v = jnp.concatenate([ehi, elo, ehi], axis=1)
    ghi, glo = _hilo(emb)
    hilo = jnp.concatenate([ghi, glo], axis=1)

    fr_hw[...] = f_hw_ref[...]
    sc_hits[...] = jnp.zeros_like(sc_hits)

    for si, pn in enumerate(_PNS):
        sq = pn * pn
        n = _B * sq
        last = si == len(_PNS) - 1

        if last:
            ch4 = 2 * _CHUNK
            def _qloop4(i, _):
                rest_a = fr_hw[pl.ds(2 * i * ch4, ch4), :]
                rest_b = fr_hw[pl.ds((2 * i + 1) * ch4, ch4), :]
                _, pa = _quant_chunk(rest_a, b96v, hilo, want_h=False)
                _, pb = _quant_chunk(rest_b, b96v, hilo, want_h=False)
                sc_hits[0:1, :] += pa + pb
                return 0
            jax.lax.fori_loop(0, _N_FULL // ch4 // 2, _qloop4, 0)
            break

        fr_cf = f_cf_ref[...] if si == 0 else _cf_of_hw(fr_hw[...])
        psq = 4 if si == 0 else sq
        pt = pts_ref[si * 2 * _S:(si + 1) * 2 * _S, 0:psq]
        fh, fl = _hilo(fr_cf)
        a512 = jnp.concatenate([fh, fl], axis=1)
        pooled_nc = _nc_of_cf(
            jax.lax.dot(a512, pt, preferred_element_type=_F32), psq)
        if pn == 1:
            r64 = jax.lax.broadcasted_iota(jnp.int32, (_B, 4 * _B), 0)
            c64 = jax.lax.broadcasted_iota(jnp.int32, (_B, 4 * _B), 1)
            m64 = jnp.where(c64 // 4 == r64, 0.25, 0.0)
            rest_nc = jax.lax.dot(m64, pooled_nc, precision=_HIGH)
        else:
            rest_nc = pooled_nc

        if n <= _CHUNK:
            h_nc, part = _quant_chunk(rest_nc, b96v, hilo, want_h=True)
            sc_hits[0:1, :] += part
        else:
            sc_h[pl.ds(0, n), :] = rest_nc

            def _qloop3(i, _):
                ra = sc_h[pl.ds(2 * i * _CHUNK, _CHUNK), :]
                rb = sc_h[pl.ds((2 * i + 1) * _CHUNK, _CHUNK), :]
                ha, pa = _quant_chunk(ra, b96v, hilo, want_h=True)
                hb, pb = _quant_chunk(rb, b96v, hilo, want_h=True)
                sc_h[pl.ds(2 * i * _CHUNK, _CHUNK), :] = ha
                sc_h[pl.ds((2 * i + 1) * _CHUNK, _CHUNK), :] = hb
                sc_hits[0:1, :] += pa + pb
                return 0
            jax.lax.fori_loop(0, n // _CHUNK // 2, _qloop3, 0)
            h_nc = sc_h[pl.ds(0, n), :]

        if pn == 1:
            hup_hw = jnp.broadcast_to(
                h_nc.reshape(_B, 1, _C), (_B, _S, _C)).reshape(_N_FULL, _C)
        else:
            ut = uts_ref[si * 192:si * 192 + 3 * sq, :]
            h_cf = _cf_of_nc(h_nc, sq)
            hh, hlo = _hilo(h_cf)
            a3 = jnp.concatenate([hh, hh, hlo], axis=1)
            hup_cf = jax.lax.dot(a3, ut, preferred_element_type=_F32)
            hup_hw = _hw_of_cf(hup_cf)
        sc_h[...] = hup_hw

        w9 = w9s_ref[si * 9 * _C:(si + 1) * 9 * _C, :].astype(_BF16)
        bias = bias_ref[si * 8:si * 8 + 1, :]

        def _cloop(j, _):
            rows = pl.ds(j * _CONV_ROWS, _CONV_ROWS)
            fr_hw[rows, :] = _conv_block(sc_h[rows, :], fr_hw[rows, :],
                                         w9, bias)
            return 0
        jax.lax.fori_loop(0, _N_FULL // _CONV_ROWS, _cloop, 0)

    hits = sc_hits[0:1, :]
    total = jnp.sum(hits)
    avg = hits / jnp.maximum(total, 1.0)
    ent = jnp.sum(avg * jnp.log(avg + 1e-10))
    f_cf = f_cf_ref[...]
    loss = 6.25 * (jnp.sum(f_cf * f_cf) / (_B * _C * _S))
    fhat_ref[...] = jnp.zeros_like(fhat_ref)
    loss_ref[...] = jnp.full((8, 128), loss, _F32)
    ppl_ref[...] = jnp.full((8, 128), jnp.exp(-ent), _F32)


def kernel(f_BChw, emb_weight, phi_w, phi_b):
    f = f_BChw.astype(_F32)
    f_cf = f.reshape(_B * _C, _S)
    f_hw = f.transpose(0, 2, 3, 1).reshape(_N_FULL, _C)

    pts = jnp.asarray(np.concatenate(
        [np.concatenate([_pool_mat_t(pn), _pool_mat_t(pn)], axis=0)
         for pn in (2, 2, 4, 8)], axis=0)).astype(_BF16)
    uts = jnp.concatenate([_upsample_stack(pn) for pn in (1, 2, 4, 8)],
                          axis=0)
    w9s = jnp.concatenate(
        [jnp.concatenate([phi_w[_PI[si], :, :, a, b].T
                          for a in range(3) for b in range(3)], axis=0)
         for si in range(4)], axis=0)
    bias = jnp.concatenate(
        [jnp.broadcast_to(phi_b[_PI[si]][None, :], (8, _C))
         for si in range(4)], axis=0)

    fhat_cf, loss_t, ppl_t = pl.pallas_call(
        _body,
        out_shape=[jax.ShapeDtypeStruct((_B * _C, _S), _F32),
                   jax.ShapeDtypeStruct((8, 128), _F32),
                   jax.ShapeDtypeStruct((8, 128), _F32)],
        scratch_shapes=[pltpu.VMEM((_N_FULL, _C), _F32),
                        pltpu.VMEM((_N_FULL, _C), _F32),
                        pltpu.VMEM((8, _VOCAB), _F32)],
    )(f_cf, f_hw, emb_weight, w9s, bias, pts, uts)

    return (fhat_cf.reshape(_B, _C, _HW, _HW), loss_t[0, 0], ppl_t[0, 0])


# speedup vs baseline: 1.1382x; 1.0221x over previous
import jax
import jax.numpy as jnp
import numpy as np
from jax.experimental import pallas as pl
from jax.experimental.pallas import tpu as pltpu

_VOCAB = 4096
_B = 64
_C = 32
_HW = 16
_S = _HW * _HW
_N_FULL = _B * _S
_PNS = (1, 2, 4, 8, 16)
_PI = (0, 1, 1, 2)
_F32 = jnp.float32
_BF16 = jnp.bfloat16
_HIGH = jax.lax.Precision.HIGHEST
_CHUNK = 256
_CONV_ROWS = 2048
_NT = (((1,), (1,)), ((), ()))


def _pool_mat_t(pn: int) -> np.ndarray:
    k = _HW // pn
    p1 = np.zeros((pn, _HW), np.float32)
    for p in range(pn):
        p1[p, p * k:(p + 1) * k] = 1.0 / k
    p2 = np.kron(p1, p1)
    out = np.zeros((_S, 64), np.float32)
    out[:, :pn * pn] = p2.T
    return out


def _upsample_stack(pn: int):
    a = jax.image.resize(jnp.eye(pn, dtype=_F32), (_HW, pn), method="bicubic")
    ut = jnp.kron(a, a).T
    hi = ut.astype(_BF16)
    lo = (ut - hi.astype(_F32)).astype(_BF16)
    sq = pn * pn
    out = jnp.zeros((192, _S), _BF16)
    out = out.at[0:sq, :].set(hi)
    out = out.at[sq:2 * sq, :].set(lo)
    out = out.at[2 * sq:3 * sq, :].set(hi)
    return out


def _hw_of_cf(x_cf):
    return jnp.swapaxes(x_cf.reshape(_B, _C, _S), 1, 2).reshape(_N_FULL, _C)


def _cf_of_hw(x_hw):
    return jnp.swapaxes(x_hw.reshape(_B, _S, _C), 1, 2).reshape(_B * _C, _S)


def _nc_of_cf(x_cf, sq):
    return jnp.swapaxes(x_cf.reshape(_B, _C, sq), 1, 2).reshape(_B * sq, _C)


def _cf_of_nc(x_nc, sq):
    return jnp.swapaxes(x_nc.reshape(_B, sq, _C), 1, 2).reshape(_B * _C, sq)


def _hilo(x):
    hi = x.astype(_BF16)
    lo = (x - hi.astype(_F32)).astype(_BF16)
    return hi, lo


def _quant_chunk(rest, b96v, hilo, want_h):
    rh, rl = _hilo(rest)
    a96 = jnp.concatenate([rh, rh, rl], axis=1)
    sims = jax.lax.dot_general(a96, b96v, _NT, preferred_element_type=_F32)
    m = jnp.max(sims, axis=1, keepdims=True)
    h = None
    if want_h:
        onehot = (sims == m).astype(_BF16)
        hl = jax.lax.dot(onehot, hilo, preferred_element_type=_F32)
        h = hl[:, :_C] + hl[:, _C:]
        part = jnp.sum(onehot.astype(_F32), axis=0, keepdims=True)
    else:
        part = jnp.sum(jnp.where(sims == m, 1.0, 0.0), axis=0, keepdims=True)
    return h, part


def _conv_block(x, fr, w9, bias):
    n = x.shape[0]
    zeros = jnp.zeros_like(x)
    ii = jax.lax.broadcasted_iota(jnp.int32, (n, 1), 0)
    w = ii & (_HW - 1)
    h = (ii >> 4) & (_HW - 1)
    taps = []
    for dh in (-1, 0, 1):
        for dw in (-1, 0, 1):
            s = dh * _HW + dw
            if s == 0:
                xs = x
            elif s > 0:
                xs = jnp.concatenate([x[s:, :], zeros[:s, :]], axis=0)
            else:
                xs = jnp.concatenate([zeros[:(-s), :], x[:s, :]], axis=0)
            okh = jnp.logical_and(h + dh >= 0, h + dh < _HW)
            okw = jnp.logical_and(w + dw >= 0, w + dw < _HW)
            taps.append(jnp.where(jnp.logical_and(okh, okw), xs, 0.0)
                        .astype(_BF16))
    y = jnp.broadcast_to(bias, (n, _C)).astype(_F32)
    y = y + jax.lax.dot(jnp.concatenate(taps[:6], axis=1), w9[:6 * _C, :],
                        preferred_element_type=_F32)
    y = y + jax.lax.dot(jnp.concatenate(taps[6:], axis=1), w9[6 * _C:, :],
                        preferred_element_type=_F32)
    return fr - 0.5 * x - 0.5 * y


def _body(f_cf_ref, f_hw_ref, emb_ref, w9s_ref, bias_ref, pts_ref, uts_ref,
          fhat_ref, loss_ref, ppl_ref,
          fr_hw, sc_h, sc_hits):
    emb = emb_ref[...]
    norm = jnp.sqrt(jnp.sum(emb * emb, axis=1, keepdims=True))
    en = emb / jnp.maximum(norm, 1e-12)
    ehi, elo = _hilo(en)
    b96v = jnp.concatenate([ehi, elo, ehi], axis=1)
    ghi, glo = _hilo(emb)
    hilo = jnp.concatenate([ghi, glo], axis=1)

    fr_hw[...] = f_hw_ref[...]
    sc_hits[...] = jnp.zeros_like(sc_hits)

    for si, pn in enumerate(_PNS):
        sq = pn * pn
        n = _B * sq
        last = si == len(_PNS) - 1

        if last:
            ch4 = 2 * _CHUNK
            def _qloop4(i, _):
                rest_a = fr_hw[pl.ds(4 * i * ch4, ch4), :]
                rest_b = fr_hw[pl.ds((4 * i + 1) * ch4, ch4), :]
                rest_c = fr_hw[pl.ds((4 * i + 2) * ch4, ch4), :]
                rest_d = fr_hw[pl.ds((4 * i + 3) * ch4, ch4), :]
                _, pa = _quant_chunk(rest_a, b96v, hilo, want_h=False)
                _, pb = _quant_chunk(rest_b, b96v, hilo, want_h=False)
                _, pc = _quant_chunk(rest_c, b96v, hilo, want_h=False)
                _, pd = _quant_chunk(rest_d, b96v, hilo, want_h=False)
                sc_hits[0:1, :] += (pa + pb) + (pc + pd)
                return 0
            jax.lax.fori_loop(0, _N_FULL // ch4 // 4, _qloop4, 0)
            break

        fr_cf = f_cf_ref[...] if si == 0 else _cf_of_hw(fr_hw[...])
        psq = 4 if si == 0 else sq
        pt = pts_ref[si * 2 * _S:(si + 1) * 2 * _S, 0:psq]
        fh, fl = _hilo(fr_cf)
        a512 = jnp.concatenate([fh, fl], axis=1)
        pooled_nc = _nc_of_cf(
            jax.lax.dot(a512, pt, preferred_element_type=_F32), psq)
        if pn == 1:
            r64 = jax.lax.broadcasted_iota(jnp.int32, (_B, 4 * _B), 0)
            c64 = jax.lax.broadcasted_iota(jnp.int32, (_B, 4 * _B), 1)
            m64 = jnp.where(c64 // 4 == r64, 0.25, 0.0)
            rest_nc = jax.lax.dot(m64, pooled_nc, precision=_HIGH)
        else:
            rest_nc = pooled_nc

        if n <= _CHUNK:
            h_nc, part = _quant_chunk(rest_nc, b96v, hilo, want_h=True)
            sc_hits[0:1, :] += part
        else:
            sc_h[pl.ds(0, n), :] = rest_nc

            def _qloop3(i, _):
                ra = sc_h[pl.ds(2 * i * _CHUNK, _CHUNK), :]
                rb = sc_h[pl.ds((2 * i + 1) * _CHUNK, _CHUNK), :]
                ha, pa = _quant_chunk(ra, b96v, hilo, want_h=True)
                hb, pb = _quant_chunk(rb, b96v, hilo, want_h=True)
                sc_h[pl.ds(2 * i * _CHUNK, _CHUNK), :] = ha
                sc_h[pl.ds((2 * i + 1) * _CHUNK, _CHUNK), :] = hb
                sc_hits[0:1, :] += pa + pb
                return 0
            jax.lax.fori_loop(0, n // _CHUNK // 2, _qloop3, 0)
            h_nc = sc_h[pl.ds(0, n), :]

        if pn == 1:
            hup_hw = jnp.broadcast_to(
                h_nc.reshape(_B, 1, _C), (_B, _S, _C)).reshape(_N_FULL, _C)
        else:
            ut = uts_ref[si * 192:si * 192 + 3 * sq, :]
            h_cf = _cf_of_nc(h_nc, sq)
            hh, hlo = _hilo(h_cf)
            a3 = jnp.concatenate([hh, hh, hlo], axis=1)
            hup_cf = jax.lax.dot(a3, ut, preferred_element_type=_F32)
            hup_hw = _hw_of_cf(hup_cf)
        sc_h[...] = hup_hw

        w9 = w9s_ref[si * 9 * _C:(si + 1) * 9 * _C, :].astype(_BF16)
        bias = bias_ref[si * 8:si * 8 + 1, :]

        def _cloop(j, _):
            rows = pl.ds(j * _CONV_ROWS, _CONV_ROWS)
            fr_hw[rows, :] = _conv_block(sc_h[rows, :], fr_hw[rows, :],
                                         w9, bias)
            return 0
        jax.lax.fori_loop(0, _N_FULL // _CONV_ROWS, _cloop, 0)

    hits = sc_hits[0:1, :]
    total = jnp.sum(hits)
    avg = hits / jnp.maximum(total, 1.0)
    ent = jnp.sum(avg * jnp.log(avg + 1e-10))
    f_cf = f_cf_ref[...]
    loss = 6.25 * (jnp.sum(f_cf * f_cf) / (_B * _C * _S))
    fhat_ref[...] = jnp.zeros_like(fhat_ref)
    loss_ref[...] = jnp.full((8, 128), loss, _F32)
    ppl_ref[...] = jnp.full((8, 128), jnp.exp(-ent), _F32)


def kernel(f_BChw, emb_weight, phi_w, phi_b):
    f = f_BChw.astype(_F32)
    f_cf = f.reshape(_B * _C, _S)
    f_hw = f.transpose(0, 2, 3, 1).reshape(_N_FULL, _C)

    pts = jnp.asarray(np.concatenate(
        [np.concatenate([_pool_mat_t(pn), _pool_mat_t(pn)], axis=0)
         for pn in (2, 2, 4, 8)], axis=0)).astype(_BF16)
    uts = jnp.concatenate([_upsample_stack(pn) for pn in (1, 2, 4, 8)],
                          axis=0)
    w9s = jnp.concatenate(
        [jnp.concatenate([phi_w[_PI[si], :, :, a, b].T
                          for a in range(3) for b in range(3)], axis=0)
         for si in range(4)], axis=0)
    bias = jnp.concatenate(
        [jnp.broadcast_to(phi_b[_PI[si]][None, :], (8, _C))
         for si in range(4)], axis=0)

    fhat_cf, loss_t, ppl_t = pl.pallas_call(
        _body,
        out_shape=[jax.ShapeDtypeStruct((_B * _C, _S), _F32),
                   jax.ShapeDtypeStruct((8, 128), _F32),
                   jax.ShapeDtypeStruct((8, 128), _F32)],
        scratch_shapes=[pltpu.VMEM((_N_FULL, _C), _F32),
                        pltpu.VMEM((_N_FULL, _C), _F32),
                        pltpu.VMEM((8, _VOCAB), _F32)],
    )(f_cf, f_hw, emb_weight, w9s, bias, pts, uts)

    return (fhat_cf.reshape(_B, _C, _HW, _HW), loss_t[0, 0], ppl_t[0, 0])
